# Initial kernel scaffold; baseline (speedup 1.0000x reference)
import jax
import jax.numpy as jnp
from jax.experimental import pallas as pl


def kernel(x, pos, edge_index, edge_attr, importance, W1, W2, Wn1, Wn2):
    raise NotImplementedError("write your pallas kernel here")


# speedup vs baseline: 1.6546x; 1.6546x over previous
import functools

import numpy as np
import jax
import jax.numpy as jnp
from jax import lax
from jax.experimental import pallas as pl
from jax.experimental.pallas import tpu as pltpu
from jax.experimental.pallas import tpu_sc as plsc

N = 10000
E = 320000
S = 8
V = 4
D_EDGE = 4
H_MSG = 64

NW = 32
EW = E // NW
CHUNK = 80
NCHUNK = EW // CHUNK
NTILE = 16
ROWS_T = N // NTILE
WCH = 125
NWCH = ROWS_T // WCH

SQRT3 = float(np.sqrt(3.0))
ALPHA = float(1.0 / np.sqrt(S + V))
INV_SQRT_NN = float(1.0 / np.sqrt(E / N))
INV_SQRT_HUPD = float(1.0 / np.sqrt(128.0))


def _build_consts():
    RSH = np.zeros((3, 12), np.float32)
    CD = np.zeros((12, 4), np.float32)
    for u in range(4):
        for i in range(3):
            RSH[i, 3 * u + i] = 1.0
            CD[3 * u + i, u] = 1.0 / np.sqrt(3.0)
    RA = np.zeros((12, 96), np.float32)
    for u in range(8):
        for v in range(8):
            RA[u, 8 * u + v] = 1.0
    for u in range(4):
        for v in range(8):
            RA[8 + u, 64 + 8 * u + v] = 1.0
    CS = np.zeros((96, 8), np.float32)
    for a in range(96):
        CS[a, a % 8] = 1.0
    R2M = np.zeros((8, 32), np.float32)
    C2M = np.zeros((32, 4), np.float32)
    for u in range(8):
        for v in range(4):
            R2M[u, 4 * u + v] = 1.0
            C2M[4 * u + v, v] = 1.0
    RG = np.zeros((4, 12), np.float32)
    RS = np.zeros((3, 12), np.float32)
    for v in range(4):
        for i in range(3):
            RG[v, 3 * v + i] = 1.0
            RS[i, 3 * v + i] = 1.0
    R4 = np.zeros((16, 48), np.float32)
    R5 = np.zeros((12, 48), np.float32)
    C3 = np.zeros((48, 12), np.float32)
    for u in range(4):
        for v in range(4):
            for i in range(3):
                R4[4 * u + v, 12 * u + 3 * v + i] = 1.0
                R5[3 * u + i, 12 * u + 3 * v + i] = 1.0
                C3[12 * u + 3 * v + i, 3 * v + i] = 1.0
    return RSH, CD, RA, CS, R2M, C2M, RG, RS, R4, R5, C3


_RSH, _CD, _RA, _CS, _R2M, _C2M, _RG, _RS, _R4, _R5, _C3 = _build_consts()


def _mm(a, b):
    return jax.lax.dot(a, b, precision=jax.lax.Precision.HIGHEST,
                       preferred_element_type=jnp.float32)


@functools.cache
def _get_gather_kernel():
    mesh = plsc.VectorSubcoreMesh(core_axis_name="c", subcore_axis_name="s")

    @functools.partial(
        pl.kernel,
        mesh=mesh,
        out_type=[jax.ShapeDtypeStruct((E, 32), jnp.float32),
                  jax.ShapeDtypeStruct((E, 8), jnp.float32)],
        scratch_types=[
            pltpu.VMEM((CHUNK,), jnp.int32),
            pltpu.VMEM((CHUNK, 32), jnp.float32),
            pltpu.VMEM((CHUNK,), jnp.int32),
            pltpu.VMEM((CHUNK, 8), jnp.float32),
            pltpu.SemaphoreType.DMA,
            pltpu.SemaphoreType.DMA,
        ],
        compiler_params=pltpu.CompilerParams(use_tc_tiling_on_sc=False),
    )
    def _gather_kernel(xt_hbm, post_hbm, src_hbm, dstidx_hbm, xs_out, pd_out,
                       sidx, srows, didx, drows, sem1, sem2):
        c = lax.axis_index("c")
        s = lax.axis_index("s")
        base = (s * 2 + c) * EW

        def step(i, carry):
            off = base + i * CHUNK
            pltpu.sync_copy(src_hbm.at[pl.ds(off, CHUNK)], sidx)
            pltpu.async_copy(xt_hbm.at[sidx], srows, sem1).wait()
            pltpu.sync_copy(srows, xs_out.at[pl.ds(off, CHUNK)])
            pltpu.sync_copy(dstidx_hbm.at[pl.ds(off, CHUNK)], didx)
            pltpu.async_copy(post_hbm.at[didx], drows, sem2).wait()
            pltpu.sync_copy(drows, pd_out.at[pl.ds(off, CHUNK)])
            return carry

        lax.fori_loop(0, NCHUNK, step, 0)

    return _gather_kernel


BE = 2000


def _edge_body(xs_ref, pd_ref, ea_ref, w1_ref, w2_ref,
               rsh_ref, cd_ref, ra_ref, cs_ref, r2m_ref, c2m_ref,
               rg_ref, rs_ref, r4_ref, r5_ref, c3_ref, out_ref):
    xs = xs_ref[...]
    pd = pd_ref[...]
    ea = ea_ref[...]
    ev = pd[:, :3] - xs[:, 20:23]
    r = jnp.sqrt(jnp.sum(ev * ev, axis=1, keepdims=True))
    unit = ev / jnp.maximum(r, 1e-12)
    sh = SQRT3 * jnp.concatenate(
        [unit[:, 1:2], unit[:, 2:3], unit[:, 0:1]], axis=1)
    h = _mm(ea, w1_ref[...]) * 0.5
    h = h * jax.nn.sigmoid(h)
    tpw = _mm(h, w2_ref[...]) * 0.125
    s1 = xs[:, :S]
    v1 = xs[:, S:20]
    dots = _mm(v1 * _mm(sh, rsh_ref[...]), cd_ref[...])
    amat = _mm(jnp.concatenate([s1, dots], axis=1), ra_ref[...])
    out_s = _mm(tpw[:, :96] * amat, cs_ref[...]) * ALPHA
    g = _mm(tpw[:, 96:128] * _mm(s1, r2m_ref[...]), c2m_ref[...])
    t1 = _mm(g, rg_ref[...]) * _mm(sh, rs_ref[...])
    t2 = _mm(_mm(tpw[:, 128:144], r4_ref[...]) *
             _mm(v1, r5_ref[...]), c3_ref[...])
    out_v = (t1 + t2) * ALPHA
    out_ref[...] = jnp.concatenate(
        [out_s, out_v, jnp.zeros((out_s.shape[0], 12), jnp.float32)], axis=1)


_CONSTS = (_RSH, _CD, _RA, _CS, _R2M, _C2M, _RG, _RS, _R4, _R5, _C3)


def _edge_in_specs(be):
    specs = [
        pl.BlockSpec((be, 32), lambda i: (i, 0)),
        pl.BlockSpec((be, 8), lambda i: (i, 0)),
        pl.BlockSpec((be, 4), lambda i: (i, 0)),
        pl.BlockSpec((D_EDGE, H_MSG), lambda i: (0, 0)),
        pl.BlockSpec((H_MSG, 144), lambda i: (0, 0)),
    ]
    for cmat in _CONSTS:
        specs.append(pl.BlockSpec(cmat.shape, lambda i: (0, 0)))
    return specs


def _edge_pallas(xs, pdst, edge_attr, W1, W2):
    return pl.pallas_call(
        _edge_body,
        grid=(E // BE,),
        in_specs=_edge_in_specs(BE),
        out_specs=pl.BlockSpec((BE, 32), lambda i: (i, 0)),
        out_shape=jax.ShapeDtypeStruct((E, 32), jnp.float32),
    )(xs, pdst, edge_attr, W1, W2, *[jnp.asarray(c) for c in _CONSTS])


@functools.cache
def _get_scatter_kernel():
    mesh = plsc.VectorSubcoreMesh(core_axis_name="c", subcore_axis_name="s")

    @functools.partial(
        pl.kernel,
        mesh=mesh,
        out_type=jax.ShapeDtypeStruct((2, N, 32), jnp.float32),
        scratch_types=[
            pltpu.VMEM((CHUNK,), jnp.int32),
            pltpu.VMEM((CHUNK, 32), jnp.float32),
            pltpu.VMEM((WCH, 32), jnp.float32),
            pltpu.VMEM_SHARED((N, 32), jnp.float32),
        ],
        compiler_params=pltpu.CompilerParams(use_tc_tiling_on_sc=False),
    )
    def _scatter_kernel(tp_hbm, dstidx_hbm, out_hbm, idxv, rowsv, bounce, acc):
        c = lax.axis_index("c")
        s = lax.axis_index("s")
        base = (s * 2 + c) * EW

        zeros16 = jnp.zeros((16,), jnp.float32)

        def zrow(r, carry):
            bounce[r, pl.ds(0, 16)] = zeros16
            bounce[r, pl.ds(16, 16)] = zeros16
            return carry

        lax.fori_loop(0, WCH, zrow, 0)

        def zchunk(k, carry):
            pltpu.sync_copy(bounce, acc.at[pl.ds(s * ROWS_T + k * WCH, WCH)])
            return carry

        lax.fori_loop(0, NWCH, zchunk, 0)
        plsc.subcore_barrier()

        def step(i, carry):
            off = base + i * CHUNK
            pltpu.sync_copy(dstidx_hbm.at[pl.ds(off, CHUNK)], idxv)
            pltpu.sync_copy(tp_hbm.at[pl.ds(off, CHUNK)], rowsv)
            pltpu.sync_copy(rowsv, acc.at[idxv], add=True)
            return carry

        lax.fori_loop(0, NCHUNK, step, 0)
        plsc.subcore_barrier()

        def wchunk(k, carry):
            r0 = s * ROWS_T + k * WCH
            pltpu.sync_copy(acc.at[pl.ds(r0, WCH)], bounce)
            pltpu.sync_copy(bounce, out_hbm.at[c, pl.ds(r0, WCH)])
            return carry

        lax.fori_loop(0, NWCH, wchunk, 0)

    return _scatter_kernel


BN = 2000


def _node_body(p0_ref, p1_ref, x_ref, imp_ref, wn1_ref, wn2_ref, out_ref):
    msgs = (p0_ref[...] + p1_ref[...])[:, :20] * (imp_ref[0, 0] * INV_SQRT_NN)
    xb = x_ref[...]
    sc_in = jnp.concatenate([msgs[:, :S], xb[:, :S]], axis=1)
    hn = _mm(sc_in, wn1_ref[...]) * 0.25
    hn = hn * jax.nn.sigmoid(hn)
    scalars = _mm(hn, wn2_ref[...]) * INV_SQRT_HUPD
    geoms = (msgs[:, S:20] + xb[:, S:20]) * 0.5
    out_ref[...] = jnp.concatenate([scalars, geoms], axis=1)


def _node_pallas(p0, p1, x, imp, Wn1, Wn2):
    return pl.pallas_call(
        _node_body,
        grid=(N // BN,),
        in_specs=[
            pl.BlockSpec((BN, 32), lambda i: (i, 0)),
            pl.BlockSpec((BN, 32), lambda i: (i, 0)),
            pl.BlockSpec((BN, 20), lambda i: (i, 0)),
            pl.BlockSpec(memory_space=pltpu.SMEM),
            pl.BlockSpec((2 * S, 128), lambda i: (0, 0)),
            pl.BlockSpec((128, S), lambda i: (0, 0)),
        ],
        out_specs=pl.BlockSpec((BN, 20), lambda i: (i, 0)),
        out_shape=jax.ShapeDtypeStruct((N, 20), jnp.float32),
    )(p0, p1, x, imp, Wn1, Wn2)


def kernel(x, pos, edge_index, edge_attr, importance, W1, W2, Wn1, Wn2):
    src = edge_index[0]
    dst = edge_index[1]
    xt = jnp.concatenate(
        [x, pos, jnp.zeros((N, 9), jnp.float32)], axis=1)
    post = jnp.concatenate(
        [pos, jnp.zeros((N, 5), jnp.float32)], axis=1)
    xs, pdst = _get_gather_kernel()(xt, post, src, dst)
    tp = _edge_pallas(xs, pdst, edge_attr, W1, W2)
    partials = _get_scatter_kernel()(tp, dst)
    imp = importance.reshape(1, 1)
    return _node_pallas(partials[0], partials[1], x, imp, Wn1, Wn2)


# speedup vs baseline: 1.8326x; 1.1076x over previous
import functools

import numpy as np
import jax
import jax.numpy as jnp
from jax import lax
from jax.experimental import pallas as pl
from jax.experimental.pallas import tpu as pltpu
from jax.experimental.pallas import tpu_sc as plsc

N = 10000
E = 320000
S = 8
V = 4
D_EDGE = 4
H_MSG = 64

NW = 32
EW = E // NW
SUB = 80
NCHUNK = EW // SUB
SB = 400
SUBS = SB // SUB
NSUP = EW // SB
NTILE = 16
ROWS_T = N // NTILE
WCH = 125
NWCH = ROWS_T // WCH

SQRT3 = float(np.sqrt(3.0))
ALPHA = float(1.0 / np.sqrt(S + V))
INV_SQRT_NN = float(1.0 / np.sqrt(E / N))
INV_SQRT_HUPD = float(1.0 / np.sqrt(128.0))


def _build_consts():
    RSH = np.zeros((3, 12), np.float32)
    CD = np.zeros((12, 4), np.float32)
    for u in range(4):
        for i in range(3):
            RSH[i, 3 * u + i] = 1.0
            CD[3 * u + i, u] = 1.0 / np.sqrt(3.0)
    RA = np.zeros((12, 96), np.float32)
    for u in range(8):
        for v in range(8):
            RA[u, 8 * u + v] = 1.0
    for u in range(4):
        for v in range(8):
            RA[8 + u, 64 + 8 * u + v] = 1.0
    CS = np.zeros((96, 8), np.float32)
    for a in range(96):
        CS[a, a % 8] = 1.0
    R2M = np.zeros((8, 32), np.float32)
    C2M = np.zeros((32, 4), np.float32)
    for u in range(8):
        for v in range(4):
            R2M[u, 4 * u + v] = 1.0
            C2M[4 * u + v, v] = 1.0
    RG = np.zeros((4, 12), np.float32)
    RS = np.zeros((3, 12), np.float32)
    for v in range(4):
        for i in range(3):
            RG[v, 3 * v + i] = 1.0
            RS[i, 3 * v + i] = 1.0
    R4 = np.zeros((16, 48), np.float32)
    R5 = np.zeros((12, 48), np.float32)
    C3 = np.zeros((48, 12), np.float32)
    for u in range(4):
        for v in range(4):
            for i in range(3):
                R4[4 * u + v, 12 * u + 3 * v + i] = 1.0
                R5[3 * u + i, 12 * u + 3 * v + i] = 1.0
                C3[12 * u + 3 * v + i, 3 * v + i] = 1.0
    return RSH, CD, RA, CS, R2M, C2M, RG, RS, R4, R5, C3


_RSH, _CD, _RA, _CS, _R2M, _C2M, _RG, _RS, _R4, _R5, _C3 = _build_consts()


def _mm(a, b):
    return jax.lax.dot(a, b, precision=jax.lax.Precision.HIGHEST,
                       preferred_element_type=jnp.float32)


@functools.cache
def _get_gather_kernel():
    mesh = plsc.VectorSubcoreMesh(core_axis_name="c", subcore_axis_name="s")

    @functools.partial(
        pl.kernel,
        mesh=mesh,
        out_type=[jax.ShapeDtypeStruct((E, 32), jnp.float32),
                  jax.ShapeDtypeStruct((E, 8), jnp.float32)],
        scratch_types=[
            pltpu.VMEM((NCHUNK, SUB), jnp.int32),
            pltpu.VMEM((NCHUNK, SUB), jnp.int32),
            pltpu.VMEM((SB, 32), jnp.float32),
            pltpu.VMEM((SB, 32), jnp.float32),
            pltpu.VMEM((SB, 8), jnp.float32),
            pltpu.VMEM((SB, 8), jnp.float32),
            pltpu.SemaphoreType.DMA,
            pltpu.SemaphoreType.DMA,
            pltpu.SemaphoreType.DMA,
            pltpu.SemaphoreType.DMA,
        ],
        compiler_params=pltpu.CompilerParams(use_tc_tiling_on_sc=False),
    )
    def _gather_kernel(xt_hbm, post_hbm, src2d_hbm, dst2d_hbm, xs_out, pd_out,
                       sidx, didx, sr0, sr1, pr0, pr1, gsem, psem, w0, w1):
        c = lax.axis_index("c")
        s = lax.axis_index("s")
        wid = s * 2 + c
        base = wid * EW
        crow = wid * NCHUNK
        pltpu.sync_copy(src2d_hbm.at[pl.ds(crow, NCHUNK)], sidx)
        pltpu.sync_copy(dst2d_hbm.at[pl.ds(crow, NCHUNK)], didx)

        def fire(g, sr, pr):
            descs = []
            for k in range(SUBS):
                row = g * SUBS + k
                descs.append(pltpu.async_copy(
                    xt_hbm.at[sidx.at[row]],
                    sr.at[pl.ds(k * SUB, SUB)], gsem))
                descs.append(pltpu.async_copy(
                    post_hbm.at[didx.at[row]],
                    pr.at[pl.ds(k * SUB, SUB)], psem))
            return descs

        def super_chunk(g, sr, pr, wsem):
            for d in fire(g, sr, pr):
                d.wait()
            off = base + g * SB
            wa = pltpu.async_copy(sr, xs_out.at[pl.ds(off, SB)], wsem)
            wb = pltpu.async_copy(pr, pd_out.at[pl.ds(off, SB)], wsem)
            return wa, wb

        def double_body(j, carry):
            wa0, wb0 = super_chunk(2 * j, sr0, pr0, w0)
            wa1, wb1 = super_chunk(2 * j + 1, sr1, pr1, w1)
            wa0.wait()
            wb0.wait()
            wa1.wait()
            wb1.wait()
            return carry

        lax.fori_loop(0, NSUP // 2, double_body, 0)
        wa, wb = super_chunk(NSUP - 1, sr0, pr0, w0)
        wa.wait()
        wb.wait()

    return _gather_kernel


BE = 2000


def _edge_body(xs_ref, pd_ref, ea_ref, w1_ref, w2_ref,
               rsh_ref, cd_ref, ra_ref, cs_ref, r2m_ref, c2m_ref,
               rg_ref, rs_ref, r4_ref, r5_ref, c3_ref, out_ref):
    xs = xs_ref[...]
    pd = pd_ref[...]
    ea = ea_ref[...]
    ev = pd[:, :3] - xs[:, 20:23]
    r = jnp.sqrt(jnp.sum(ev * ev, axis=1, keepdims=True))
    unit = ev / jnp.maximum(r, 1e-12)
    sh = SQRT3 * jnp.concatenate(
        [unit[:, 1:2], unit[:, 2:3], unit[:, 0:1]], axis=1)
    h = _mm(ea, w1_ref[...]) * 0.5
    h = h * jax.nn.sigmoid(h)
    tpw = _mm(h, w2_ref[...]) * 0.125
    s1 = xs[:, :S]
    v1 = xs[:, S:20]
    dots = _mm(v1 * _mm(sh, rsh_ref[...]), cd_ref[...])
    amat = _mm(jnp.concatenate([s1, dots], axis=1), ra_ref[...])
    out_s = _mm(tpw[:, :96] * amat, cs_ref[...]) * ALPHA
    g = _mm(tpw[:, 96:128] * _mm(s1, r2m_ref[...]), c2m_ref[...])
    t1 = _mm(g, rg_ref[...]) * _mm(sh, rs_ref[...])
    t2 = _mm(_mm(tpw[:, 128:144], r4_ref[...]) *
             _mm(v1, r5_ref[...]), c3_ref[...])
    out_v = (t1 + t2) * ALPHA
    out_ref[...] = jnp.concatenate(
        [out_s, out_v, jnp.zeros((out_s.shape[0], 12), jnp.float32)], axis=1)


_CONSTS = (_RSH, _CD, _RA, _CS, _R2M, _C2M, _RG, _RS, _R4, _R5, _C3)


def _edge_in_specs(be):
    specs = [
        pl.BlockSpec((be, 32), lambda i: (i, 0)),
        pl.BlockSpec((be, 8), lambda i: (i, 0)),
        pl.BlockSpec((be, 4), lambda i: (i, 0)),
        pl.BlockSpec((D_EDGE, H_MSG), lambda i: (0, 0)),
        pl.BlockSpec((H_MSG, 144), lambda i: (0, 0)),
    ]
    for cmat in _CONSTS:
        specs.append(pl.BlockSpec(cmat.shape, lambda i: (0, 0)))
    return specs


def _edge_pallas(xs, pdst, edge_attr, W1, W2):
    return pl.pallas_call(
        _edge_body,
        grid=(E // BE,),
        in_specs=_edge_in_specs(BE),
        out_specs=pl.BlockSpec((BE, 32), lambda i: (i, 0)),
        out_shape=jax.ShapeDtypeStruct((E, 32), jnp.float32),
    )(xs, pdst, edge_attr, W1, W2, *[jnp.asarray(c) for c in _CONSTS])


@functools.cache
def _get_scatter_kernel():
    mesh = plsc.VectorSubcoreMesh(core_axis_name="c", subcore_axis_name="s")

    @functools.partial(
        pl.kernel,
        mesh=mesh,
        out_type=jax.ShapeDtypeStruct((2, N, 32), jnp.float32),
        scratch_types=[
            pltpu.VMEM((NCHUNK, SUB), jnp.int32),
            pltpu.VMEM((SB, 32), jnp.float32),
            pltpu.VMEM((SB, 32), jnp.float32),
            pltpu.VMEM((WCH, 32), jnp.float32),
            pltpu.VMEM_SHARED((N, 32), jnp.float32),
            pltpu.SemaphoreType.DMA,
            pltpu.SemaphoreType.DMA,
            pltpu.SemaphoreType.DMA,
            pltpu.SemaphoreType.DMA,
        ],
        compiler_params=pltpu.CompilerParams(use_tc_tiling_on_sc=False),
    )
    def _scatter_kernel(tp_hbm, dst2d_hbm, out_hbm, didx, r0buf, r1buf,
                        bounce, acc, l0, l1, a0, a1):
        c = lax.axis_index("c")
        s = lax.axis_index("s")
        wid = s * 2 + c
        base = wid * EW
        crow = wid * NCHUNK
        pltpu.sync_copy(dst2d_hbm.at[pl.ds(crow, NCHUNK)], didx)

        zeros16 = jnp.zeros((16,), jnp.float32)

        def zrow(r, carry):
            bounce[r, pl.ds(0, 16)] = zeros16
            bounce[r, pl.ds(16, 16)] = zeros16
            return carry

        lax.fori_loop(0, WCH, zrow, 0)

        def zchunk(k, carry):
            pltpu.sync_copy(bounce, acc.at[pl.ds(s * ROWS_T + k * WCH, WCH)])
            return carry

        lax.fori_loop(0, NWCH, zchunk, 0)
        plsc.subcore_barrier()

        def load(g, rbuf, lsem):
            return pltpu.async_copy(
                tp_hbm.at[pl.ds(base + g * SB, SB)], rbuf, lsem)

        def adds(g, rbuf, asem):
            descs = []
            for k in range(SUBS):
                descs.append(pltpu.async_copy(
                    rbuf.at[pl.ds(k * SUB, SUB)],
                    acc.at[didx.at[g * SUBS + k]], asem, add=True))
            return descs

        def double_body(j, carry):
            ld0 = load(2 * j, r0buf, l0)
            ld1 = load(2 * j + 1, r1buf, l1)
            ld0.wait()
            d0 = adds(2 * j, r0buf, a0)
            ld1.wait()
            d1 = adds(2 * j + 1, r1buf, a1)
            for d in d0:
                d.wait()
            for d in d1:
                d.wait()
            return carry

        lax.fori_loop(0, NSUP // 2, double_body, 0)
        load(NSUP - 1, r0buf, l0).wait()
        for d in adds(NSUP - 1, r0buf, a0):
            d.wait()
        plsc.subcore_barrier()

        def wchunk(k, carry):
            r0 = s * ROWS_T + k * WCH
            pltpu.sync_copy(acc.at[pl.ds(r0, WCH)], bounce)
            pltpu.sync_copy(bounce, out_hbm.at[c, pl.ds(r0, WCH)])
            return carry

        lax.fori_loop(0, NWCH, wchunk, 0)

    return _scatter_kernel


BN = 2000


def _node_body(p0_ref, p1_ref, x_ref, imp_ref, wn1_ref, wn2_ref, out_ref):
    msgs = (p0_ref[...] + p1_ref[...])[:, :20] * (imp_ref[0, 0] * INV_SQRT_NN)
    xb = x_ref[...]
    sc_in = jnp.concatenate([msgs[:, :S], xb[:, :S]], axis=1)
    hn = _mm(sc_in, wn1_ref[...]) * 0.25
    hn = hn * jax.nn.sigmoid(hn)
    scalars = _mm(hn, wn2_ref[...]) * INV_SQRT_HUPD
    geoms = (msgs[:, S:20] + xb[:, S:20]) * 0.5
    out_ref[...] = jnp.concatenate([scalars, geoms], axis=1)


def _node_pallas(p0, p1, x, imp, Wn1, Wn2):
    return pl.pallas_call(
        _node_body,
        grid=(N // BN,),
        in_specs=[
            pl.BlockSpec((BN, 32), lambda i: (i, 0)),
            pl.BlockSpec((BN, 32), lambda i: (i, 0)),
            pl.BlockSpec((BN, 20), lambda i: (i, 0)),
            pl.BlockSpec(memory_space=pltpu.SMEM),
            pl.BlockSpec((2 * S, 128), lambda i: (0, 0)),
            pl.BlockSpec((128, S), lambda i: (0, 0)),
        ],
        out_specs=pl.BlockSpec((BN, 20), lambda i: (i, 0)),
        out_shape=jax.ShapeDtypeStruct((N, 20), jnp.float32),
    )(p0, p1, x, imp, Wn1, Wn2)


def kernel(x, pos, edge_index, edge_attr, importance, W1, W2, Wn1, Wn2):
    src2d = edge_index[0].reshape(E // SUB, SUB)
    dst2d = edge_index[1].reshape(E // SUB, SUB)
    xt = jnp.concatenate(
        [x, pos, jnp.zeros((N, 9), jnp.float32)], axis=1)
    post = jnp.concatenate(
        [pos, jnp.zeros((N, 5), jnp.float32)], axis=1)
    xs, pdst = _get_gather_kernel()(xt, post, src2d, dst2d)
    tp = _edge_pallas(xs, pdst, edge_attr, W1, W2)
    partials = _get_scatter_kernel()(tp, dst2d)
    imp = importance.reshape(1, 1)
    return _node_pallas(partials[0], partials[1], x, imp, Wn1, Wn2)


# speedup vs baseline: 6.1008x; 3.3291x over previous
import functools

import numpy as np
import jax
import jax.numpy as jnp
from jax import lax
from jax.experimental import pallas as pl
from jax.experimental.pallas import tpu as pltpu
from jax.experimental.pallas import tpu_sc as plsc

N = 10000
E = 320000
S = 8
V = 4
D_EDGE = 4
H_MSG = 64

NW = 32
EW = E // NW
SUB = 80
NCHUNK = EW // SUB
SB = 400
SUBS = SB // SUB
NSUP = EW // SB
NTILE = 16
ROWS_T = N // NTILE
WCH = 125
NWCH = ROWS_T // WCH

SQRT3 = float(np.sqrt(3.0))
ALPHA = float(1.0 / np.sqrt(S + V))
INV_SQRT_NN = float(1.0 / np.sqrt(E / N))
INV_SQRT_HUPD = float(1.0 / np.sqrt(128.0))


def _build_consts():
    RSH = np.zeros((3, 12), np.float32)
    CD = np.zeros((12, 4), np.float32)
    for u in range(4):
        for i in range(3):
            RSH[i, 3 * u + i] = 1.0
            CD[3 * u + i, u] = 1.0 / np.sqrt(3.0)
    RA = np.zeros((12, 96), np.float32)
    for u in range(8):
        for v in range(8):
            RA[u, 8 * u + v] = 1.0
    for u in range(4):
        for v in range(8):
            RA[8 + u, 64 + 8 * u + v] = 1.0
    CS = np.zeros((96, 8), np.float32)
    for a in range(96):
        CS[a, a % 8] = 1.0
    R2M = np.zeros((8, 32), np.float32)
    C2M = np.zeros((32, 4), np.float32)
    for u in range(8):
        for v in range(4):
            R2M[u, 4 * u + v] = 1.0
            C2M[4 * u + v, v] = 1.0
    RG = np.zeros((4, 12), np.float32)
    RS = np.zeros((3, 12), np.float32)
    for v in range(4):
        for i in range(3):
            RG[v, 3 * v + i] = 1.0
            RS[i, 3 * v + i] = 1.0
    R4 = np.zeros((16, 48), np.float32)
    R5 = np.zeros((12, 48), np.float32)
    C3 = np.zeros((48, 12), np.float32)
    for u in range(4):
        for v in range(4):
            for i in range(3):
                R4[4 * u + v, 12 * u + 3 * v + i] = 1.0
                R5[3 * u + i, 12 * u + 3 * v + i] = 1.0
                C3[12 * u + 3 * v + i, 3 * v + i] = 1.0
    return RSH, CD, RA, CS, R2M, C2M, RG, RS, R4, R5, C3


_RSH, _CD, _RA, _CS, _R2M, _C2M, _RG, _RS, _R4, _R5, _C3 = _build_consts()


def _mm(a, b):
    return jax.lax.dot(a, b, precision=jax.lax.Precision.DEFAULT,
                       preferred_element_type=jnp.float32)


@functools.cache
def _get_gather_kernel():
    mesh = plsc.VectorSubcoreMesh(core_axis_name="c", subcore_axis_name="s")

    @functools.partial(
        pl.kernel,
        mesh=mesh,
        out_type=[jax.ShapeDtypeStruct((E, 32), jnp.float32),
                  jax.ShapeDtypeStruct((E, 32), jnp.float32)],
        scratch_types=[
            pltpu.VMEM((NCHUNK, SUB), jnp.int32),
            pltpu.VMEM((NCHUNK, SUB), jnp.int32),
            pltpu.VMEM((SB, 32), jnp.float32),
            pltpu.VMEM((SB, 32), jnp.float32),
            pltpu.VMEM((SB, 32), jnp.float32),
            pltpu.VMEM((SB, 32), jnp.float32),
            pltpu.SemaphoreType.DMA,
            pltpu.SemaphoreType.DMA,
            pltpu.SemaphoreType.DMA,
            pltpu.SemaphoreType.DMA,
        ],
        compiler_params=pltpu.CompilerParams(use_tc_tiling_on_sc=False),
    )
    def _gather_kernel(xt_hbm, src2d_hbm, dst2d_hbm, xs_out, xd_out,
                       sidx, didx, sr0, sr1, dr0, dr1, gsem, psem, w0, w1):
        c = lax.axis_index("c")
        s = lax.axis_index("s")
        wid = s * 2 + c
        base = wid * EW
        crow = wid * NCHUNK
        pltpu.sync_copy(src2d_hbm.at[pl.ds(crow, NCHUNK)], sidx)
        pltpu.sync_copy(dst2d_hbm.at[pl.ds(crow, NCHUNK)], didx)

        def fire(g, sr, dr):
            descs = []
            for k in range(SUBS):
                row = g * SUBS + k
                descs.append(pltpu.async_copy(
                    xt_hbm.at[sidx.at[row]],
                    sr.at[pl.ds(k * SUB, SUB)], gsem))
                descs.append(pltpu.async_copy(
                    xt_hbm.at[didx.at[row]],
                    dr.at[pl.ds(k * SUB, SUB)], psem))
            return descs

        def super_chunk(g, sr, dr, wsem):
            for d in fire(g, sr, dr):
                d.wait()
            off = base + g * SB
            wa = pltpu.async_copy(sr, xs_out.at[pl.ds(off, SB)], wsem)
            wb = pltpu.async_copy(dr, xd_out.at[pl.ds(off, SB)], wsem)
            return wa, wb

        def double_body(j, carry):
            wa0, wb0 = super_chunk(2 * j, sr0, dr0, w0)
            wa1, wb1 = super_chunk(2 * j + 1, sr1, dr1, w1)
            wa0.wait()
            wb0.wait()
            wa1.wait()
            wb1.wait()
            return carry

        lax.fori_loop(0, NSUP // 2, double_body, 0)
        wa, wb = super_chunk(NSUP - 1, sr0, dr0, w0)
        wa.wait()
        wb.wait()

    return _gather_kernel


BE = 6400
BEQ = BE // 4


def _edge_subset(xs, xd, ea_t, w1, w2, crefs):
    (rsh, cd, ra, cs, r2m, c2m, rg, rs, r4, r5, c3) = crefs
    ev = xd[:, 20:23] - xs[:, 20:23]
    r = jnp.sqrt(jnp.sum(ev * ev, axis=1, keepdims=True))
    unit = ev / jnp.maximum(r, 1e-12)
    sh = SQRT3 * jnp.concatenate(
        [unit[:, 1:2], unit[:, 2:3], unit[:, 0:1]], axis=1)
    h = jax.lax.dot_general(
        ea_t, w1, (((0,), (0,)), ((), ())),
        precision=jax.lax.Precision.DEFAULT,
        preferred_element_type=jnp.float32) * 0.5
    h = h * jax.nn.sigmoid(h)
    tpw = _mm(h, w2) * 0.125
    s1 = xs[:, :S]
    v1 = xs[:, S:20]
    dots = _mm(v1 * _mm(sh, rsh), cd)
    amat = _mm(jnp.concatenate([s1, dots], axis=1), ra)
    out_s = _mm(tpw[:, :96] * amat, cs) * ALPHA
    g = _mm(tpw[:, 96:128] * _mm(s1, r2m), c2m)
    t1 = _mm(g, rg) * _mm(sh, rs)
    t2 = _mm(_mm(tpw[:, 128:144], r4) * _mm(v1, r5), c3)
    out_v = (t1 + t2) * ALPHA
    return jnp.concatenate(
        [out_s, out_v, jnp.zeros((out_s.shape[0], 12), jnp.float32)], axis=1)


def _edge_body(xs_ref, xd_ref, ea_ref, w1_ref, w2_ref,
               rsh_ref, cd_ref, ra_ref, cs_ref, r2m_ref, c2m_ref,
               rg_ref, rs_ref, r4_ref, r5_ref, c3_ref, out_ref):
    xsp = xs_ref[...]
    xdp = xd_ref[...]
    eap = ea_ref[...]
    w1 = w1_ref[...]
    w2 = w2_ref[...]
    crefs = (rsh_ref[...], cd_ref[...], ra_ref[...], cs_ref[...],
             r2m_ref[...], c2m_ref[...], rg_ref[...], rs_ref[...],
             r4_ref[...], r5_ref[...], c3_ref[...])
    nq = xsp.shape[0]
    outs = []
    for j in range(4):
        o = 32 * j
        outs.append(_edge_subset(
            xsp[:, o:o + 32], xdp[:, o:o + 32],
            eap[:, j * nq:(j + 1) * nq], w1, w2, crefs))
    out_ref[...] = jnp.concatenate(outs, axis=1)


_CONSTS = (_RSH, _CD, _RA, _CS, _R2M, _C2M, _RG, _RS, _R4, _R5, _C3)


def _edge_perm():
    q = np.arange(E, dtype=np.int64)
    rr = q // 4
    j = q % 4
    b = rr // BEQ
    r = rr % BEQ
    return (b * BE + j * BEQ + r).astype(np.int32)


_PERM = _edge_perm()


def _edge_in_specs(beq):
    specs = [
        pl.BlockSpec((beq, 128), lambda i: (i, 0)),
        pl.BlockSpec((beq, 128), lambda i: (i, 0)),
        pl.BlockSpec((4, 4 * beq), lambda i: (0, i)),
        pl.BlockSpec((D_EDGE, H_MSG), lambda i: (0, 0)),
        pl.BlockSpec((H_MSG, 144), lambda i: (0, 0)),
    ]
    for cmat in _CONSTS:
        specs.append(pl.BlockSpec(cmat.shape, lambda i: (0, 0)))
    return specs


def _edge_pallas(xs, xd, edge_attr, W1, W2):
    return pl.pallas_call(
        _edge_body,
        grid=(E // BE,),
        in_specs=_edge_in_specs(BEQ),
        out_specs=pl.BlockSpec((BEQ, 128), lambda i: (i, 0)),
        out_shape=jax.ShapeDtypeStruct((E // 4, 128), jnp.float32),
    )(xs.reshape(E // 4, 128), xd.reshape(E // 4, 128),
      edge_attr.T, W1, W2,
      *[jnp.asarray(c) for c in _CONSTS])


@functools.cache
def _get_scatter_kernel():
    mesh = plsc.VectorSubcoreMesh(core_axis_name="c", subcore_axis_name="s")

    @functools.partial(
        pl.kernel,
        mesh=mesh,
        out_type=jax.ShapeDtypeStruct((2, N, 32), jnp.float32),
        scratch_types=[
            pltpu.VMEM((NCHUNK, SUB), jnp.int32),
            pltpu.VMEM((SB, 32), jnp.float32),
            pltpu.VMEM((SB, 32), jnp.float32),
            pltpu.VMEM((WCH, 32), jnp.float32),
            pltpu.VMEM_SHARED((N, 32), jnp.float32),
            pltpu.SemaphoreType.DMA,
            pltpu.SemaphoreType.DMA,
            pltpu.SemaphoreType.DMA,
            pltpu.SemaphoreType.DMA,
        ],
        compiler_params=pltpu.CompilerParams(use_tc_tiling_on_sc=False),
    )
    def _scatter_kernel(tp_hbm, dst2d_hbm, out_hbm, didx, r0buf, r1buf,
                        bounce, acc, l0, l1, a0, a1):
        c = lax.axis_index("c")
        s = lax.axis_index("s")
        wid = s * 2 + c
        base = wid * EW
        crow = wid * NCHUNK
        pltpu.sync_copy(dst2d_hbm.at[pl.ds(crow, NCHUNK)], didx)

        zeros16 = jnp.zeros((16,), jnp.float32)

        def zrow(r, carry):
            bounce[r, pl.ds(0, 16)] = zeros16
            bounce[r, pl.ds(16, 16)] = zeros16
            return carry

        lax.fori_loop(0, WCH, zrow, 0)

        def zchunk(k, carry):
            pltpu.sync_copy(bounce, acc.at[pl.ds(s * ROWS_T + k * WCH, WCH)])
            return carry

        lax.fori_loop(0, NWCH, zchunk, 0)
        plsc.subcore_barrier()

        def load(g, rbuf, lsem):
            return pltpu.async_copy(
                tp_hbm.at[pl.ds(base + g * SB, SB)], rbuf, lsem)

        def adds(g, rbuf, asem):
            descs = []
            for k in range(SUBS):
                descs.append(pltpu.async_copy(
                    rbuf.at[pl.ds(k * SUB, SUB)],
                    acc.at[didx.at[g * SUBS + k]], asem, add=True))
            return descs

        def double_body(j, carry):
            ld0 = load(2 * j, r0buf, l0)
            ld1 = load(2 * j + 1, r1buf, l1)
            ld0.wait()
            d0 = adds(2 * j, r0buf, a0)
            ld1.wait()
            d1 = adds(2 * j + 1, r1buf, a1)
            for d in d0:
                d.wait()
            for d in d1:
                d.wait()
            return carry

        lax.fori_loop(0, NSUP // 2, double_body, 0)
        load(NSUP - 1, r0buf, l0).wait()
        for d in adds(NSUP - 1, r0buf, a0):
            d.wait()
        plsc.subcore_barrier()

        def wchunk(k, carry):
            r0 = s * ROWS_T + k * WCH
            pltpu.sync_copy(acc.at[pl.ds(r0, WCH)], bounce)
            pltpu.sync_copy(bounce, out_hbm.at[c, pl.ds(r0, WCH)])
            return carry

        lax.fori_loop(0, NWCH, wchunk, 0)

    return _scatter_kernel


BN = 2000


def _node_body(p0_ref, p1_ref, x_ref, imp_ref, wn1_ref, wn2_ref, out_ref):
    msgs = (p0_ref[...] + p1_ref[...])[:, :20] * (imp_ref[0, 0] * INV_SQRT_NN)
    xb = x_ref[...]
    sc_in = jnp.concatenate([msgs[:, :S], xb[:, :S]], axis=1)
    hn = _mm(sc_in, wn1_ref[...]) * 0.25
    hn = hn * jax.nn.sigmoid(hn)
    scalars = _mm(hn, wn2_ref[...]) * INV_SQRT_HUPD
    geoms = (msgs[:, S:20] + xb[:, S:20]) * 0.5
    out_ref[...] = jnp.concatenate([scalars, geoms], axis=1)


def _node_pallas(p0, p1, x, imp, Wn1, Wn2):
    return pl.pallas_call(
        _node_body,
        grid=(N // BN,),
        in_specs=[
            pl.BlockSpec((BN, 32), lambda i: (i, 0)),
            pl.BlockSpec((BN, 32), lambda i: (i, 0)),
            pl.BlockSpec((BN, 20), lambda i: (i, 0)),
            pl.BlockSpec(memory_space=pltpu.SMEM),
            pl.BlockSpec((2 * S, 128), lambda i: (0, 0)),
            pl.BlockSpec((128, S), lambda i: (0, 0)),
        ],
        out_specs=pl.BlockSpec((BN, 20), lambda i: (i, 0)),
        out_shape=jax.ShapeDtypeStruct((N, 20), jnp.float32),
    )(p0, p1, x, imp, Wn1, Wn2)


def kernel(x, pos, edge_index, edge_attr, importance, W1, W2, Wn1, Wn2):
    perm = jnp.asarray(_PERM)
    src2d = jnp.take(edge_index[0], perm).reshape(E // SUB, SUB)
    dst2d = jnp.take(edge_index[1], perm).reshape(E // SUB, SUB)
    xt = jnp.concatenate(
        [x, pos, jnp.zeros((N, 9), jnp.float32)], axis=1)
    xs, xd = _get_gather_kernel()(xt, src2d, dst2d)
    tp = _edge_pallas(xs, xd, edge_attr, W1, W2).reshape(E, 32)
    partials = _get_scatter_kernel()(tp, dst2d)
    imp = importance.reshape(1, 1)
    return _node_pallas(partials[0], partials[1], x, imp, Wn1, Wn2)
